# baseline (speedup 1.0000x reference)
import functools
import jax
import jax.numpy as jnp
from jax import lax
from jax.experimental import pallas as pl
from jax.experimental.pallas import tpu as pltpu
from jax.experimental.pallas import tpu_sc as plsc

RESO = 128
L = 16
NC = 2
NS = 16
NW = NC * NS
CHUNK = 4096
GROUPS = CHUNK // L


def _quadric_grid_sc(P_pad):
    n_chunks = P_pad // CHUNK
    s_max = n_chunks // NW
    mesh = plsc.VectorSubcoreMesh(core_axis_name="c", subcore_axis_name="s",
                                  num_cores=NC, num_subcores=NS)

    @functools.partial(
        pl.kernel,
        out_type=(
            jax.ShapeDtypeStruct((P_pad,), jnp.float32),
            jax.ShapeDtypeStruct((3 * P_pad,), jnp.float32),
        ),
        mesh=mesh,
        compiler_params=pltpu.CompilerParams(needs_layout_passes=False),
        scratch_types=dict(
            xl=pltpu.VMEM((RESO,), jnp.float32),
            yl=pltpu.VMEM((RESO,), jnp.float32),
            zl=pltpu.VMEM((RESO,), jnp.float32),
            off=pltpu.VMEM((4 * L,), jnp.float32),
            idx_v=pltpu.VMEM((CHUNK,), jnp.int32),
            pts_v=pltpu.VMEM((3 * CHUNK,), jnp.float32),
            sdf_v=pltpu.VMEM((CHUNK,), jnp.float32),
            nrm_v=pltpu.VMEM((3 * CHUNK,), jnp.float32),
        ),
    )
    def k(rpts_hbm, ridx_hbm, spts_hbm, sidx_hbm, xl_hbm, yl_hbm, zl_hbm,
          off_hbm, sdf_out_hbm, nrm_out_hbm,
          xl, yl, zl, off, idx_v, pts_v, sdf_v, nrm_v):
        wid = lax.axis_index("s") * NC + lax.axis_index("c")

        pltpu.sync_copy(xl_hbm, xl)
        pltpu.sync_copy(yl_hbm, yl)
        pltpu.sync_copy(zl_hbm, zl)
        pltpu.sync_copy(off_hbm, off)

        d = off[pl.ds(0 * L, L)]
        e = off[pl.ds(1 * L, L)]
        f = off[pl.ds(2 * L, L)]
        g = off[pl.ds(3 * L, L)]
        lane3 = lax.iota(jnp.int32, L) * 3

        def compute_chunk(want_normal):
            @pl.loop(0, GROUPS)
            def _(grp):
                base16 = grp * L
                idx = idx_v[pl.ds(base16, L)]
                ii = lax.shift_right_logical(idx, 14)
                jj = lax.shift_right_logical(idx, 7) & (RESO - 1)
                kk = idx & (RESO - 1)
                a = plsc.load_gather(xl, [ii])
                b = plsc.load_gather(yl, [jj])
                c = plsc.load_gather(zl, [kk])
                p3 = base16 * 3 + lane3
                x = plsc.load_gather(pts_v, [p3])
                y = plsc.load_gather(pts_v, [p3 + 1])
                z = plsc.load_gather(pts_v, [p3 + 2])
                if want_normal:
                    nx = (a + a) * x + d
                    ny = (b + b) * y + e
                    nz = (c + c) * z + f
                    plsc.store_scatter(nrm_v, [p3], nx)
                    plsc.store_scatter(nrm_v, [p3 + 1], ny)
                    plsc.store_scatter(nrm_v, [p3 + 2], nz)
                else:
                    sdf = (a * x * x + b * y * y + c * z * z
                           + d * x + e * y + f * z + g)
                    sdf_v[pl.ds(base16, L)] = sdf

        for s in range(s_max):
            base = (wid * s_max + s) * CHUNK
            pltpu.sync_copy(sidx_hbm.at[pl.ds(base, CHUNK)], idx_v)
            pltpu.sync_copy(spts_hbm.at[pl.ds(base * 3, 3 * CHUNK)], pts_v)
            compute_chunk(False)
            pltpu.sync_copy(sdf_v, sdf_out_hbm.at[pl.ds(base, CHUNK)])
            pltpu.sync_copy(ridx_hbm.at[pl.ds(base, CHUNK)], idx_v)
            pltpu.sync_copy(rpts_hbm.at[pl.ds(base * 3, 3 * CHUNK)], pts_v)
            compute_chunk(True)
            pltpu.sync_copy(nrm_v, nrm_out_hbm.at[pl.ds(base * 3, 3 * CHUNK)])

    return k


def _pad_to(arr, n):
    p = n - arr.shape[0]
    if p == 0:
        return arr
    return jnp.concatenate([arr, jnp.zeros((p,), arr.dtype)])


@jax.jit
def kernel(renderPointList, renderIndexList, sdfPointList, sdfIndexList,
           xLayer, yLayer, zLayer, offset):
    P = renderPointList.shape[0]
    work = NW * CHUNK
    P_pad = -(-P // work) * work
    k = _quadric_grid_sc(P_pad)
    off64 = jnp.repeat(offset, L)
    sdf, nrm = k(_pad_to(renderPointList.reshape(-1), 3 * P_pad),
                 _pad_to(renderIndexList, P_pad),
                 _pad_to(sdfPointList.reshape(-1), 3 * P_pad),
                 _pad_to(sdfIndexList, P_pad),
                 xLayer, yLayer, zLayer, off64)
    return sdf[:P], nrm[:3 * P].reshape(P, 3)


# speedup vs baseline: 1.8408x; 1.8408x over previous
import functools
import jax
import jax.numpy as jnp
from jax import lax
from jax.experimental import pallas as pl
from jax.experimental.pallas import tpu as pltpu
from jax.experimental.pallas import tpu_sc as plsc

RESO = 128
L = 16
NC = 2
NS = 16
NW = NC * NS
CHUNK = 4096
GROUPS = CHUNK // L

BR = 2048
B1 = 131072


def _tc_pad4(pts, P_pad):
    P = pts.shape[0]
    nin = -(-P // BR)

    def body(i_ref, o_ref):
        o_ref[:, 0:3] = i_ref[...]
        o_ref[:, 3:4] = jnp.zeros((BR, 1), jnp.float32)

    out = pl.pallas_call(
        body,
        grid=(P_pad // BR,),
        in_specs=[pl.BlockSpec((BR, 3), lambda i: (jnp.minimum(i, nin - 1), 0))],
        out_specs=pl.BlockSpec((BR, 4), lambda i: (i, 0)),
        out_shape=jax.ShapeDtypeStruct((P_pad, 4), jnp.float32),
    )(pts)
    return out.reshape(-1)


def _tc_pad1(idx, P_pad):
    P = idx.shape[0]
    nin = -(-P // B1)

    def body(i_ref, o_ref):
        o_ref[...] = i_ref[...]

    return pl.pallas_call(
        body,
        grid=(P_pad // B1,),
        in_specs=[pl.BlockSpec((B1,), lambda i: (jnp.minimum(i, nin - 1),))],
        out_specs=pl.BlockSpec((B1,), lambda i: (i,)),
        out_shape=jax.ShapeDtypeStruct((P_pad,), jnp.int32),
    )(idx)


def _tc_slice1(x, P):
    def body(i_ref, o_ref):
        o_ref[...] = i_ref[...]

    return pl.pallas_call(
        body,
        grid=(-(-P // B1),),
        in_specs=[pl.BlockSpec((B1,), lambda i: (i,))],
        out_specs=pl.BlockSpec((B1,), lambda i: (i,)),
        out_shape=jax.ShapeDtypeStruct((P,), jnp.float32),
    )(x)


def _tc_unpack3(n4_flat, P_pad, P):
    n4 = n4_flat.reshape(P_pad, 4)

    def body(i_ref, o_ref):
        o_ref[...] = i_ref[:, 0:3]

    return pl.pallas_call(
        body,
        grid=(-(-P // BR),),
        in_specs=[pl.BlockSpec((BR, 4), lambda i: (i, 0))],
        out_specs=pl.BlockSpec((BR, 3), lambda i: (i, 0)),
        out_shape=jax.ShapeDtypeStruct((P, 3), jnp.float32),
    )(n4)


def _quadric_grid_sc(P_pad):
    s_max = P_pad // (NW * CHUNK)
    mesh = plsc.VectorSubcoreMesh(core_axis_name="c", subcore_axis_name="s",
                                  num_cores=NC, num_subcores=NS)

    @functools.partial(
        pl.kernel,
        out_type=(
            jax.ShapeDtypeStruct((P_pad,), jnp.float32),
            jax.ShapeDtypeStruct((4 * P_pad,), jnp.float32),
        ),
        mesh=mesh,
        compiler_params=pltpu.CompilerParams(needs_layout_passes=False),
        scratch_types=dict(
            xl=pltpu.VMEM((RESO,), jnp.float32),
            yl=pltpu.VMEM((RESO,), jnp.float32),
            zl=pltpu.VMEM((RESO,), jnp.float32),
            off=pltpu.VMEM((RESO,), jnp.float32),
            idx_v=pltpu.VMEM((CHUNK,), jnp.int32),
            pts_v=pltpu.VMEM((4 * CHUNK,), jnp.float32),
            sdf_v=pltpu.VMEM((CHUNK,), jnp.float32),
            nrm_v=pltpu.VMEM((4 * CHUNK,), jnp.float32),
        ),
    )
    def k(rpts_hbm, ridx_hbm, spts_hbm, sidx_hbm, xl_hbm, yl_hbm, zl_hbm,
          off_hbm, sdf_out_hbm, nrm_out_hbm,
          xl, yl, zl, off, idx_v, pts_v, sdf_v, nrm_v):
        wid = lax.axis_index("s") * NC + lax.axis_index("c")

        pltpu.sync_copy(xl_hbm, xl)
        pltpu.sync_copy(yl_hbm, yl)
        pltpu.sync_copy(zl_hbm, zl)
        pltpu.sync_copy(off_hbm, off)

        d = off[pl.ds(0 * 2 * L, L)]
        e = off[pl.ds(1 * 2 * L, L)]
        f = off[pl.ds(2 * 2 * L, L)]
        g = off[pl.ds(3 * 2 * L, L)]
        lane4 = lax.iota(jnp.int32, L) * 4

        def compute_chunk(want_normal):
            @pl.loop(0, GROUPS)
            def _(grp):
                base16 = grp * L
                p4 = base16 * 4 + lane4
                idx = idx_v[pl.ds(base16, L)]
                ii = lax.shift_right_logical(idx, 14) & (RESO - 1)
                jj = lax.shift_right_logical(idx, 7) & (RESO - 1)
                kk = idx & (RESO - 1)
                a = plsc.load_gather(xl, [ii])
                b = plsc.load_gather(yl, [jj])
                c = plsc.load_gather(zl, [kk])
                x = plsc.load_gather(pts_v, [p4])
                y = plsc.load_gather(pts_v, [p4 + 1])
                z = plsc.load_gather(pts_v, [p4 + 2])
                if want_normal:
                    nx = (a + a) * x + d
                    ny = (b + b) * y + e
                    nz = (c + c) * z + f
                    plsc.store_scatter(nrm_v, [p4], nx)
                    plsc.store_scatter(nrm_v, [p4 + 1], ny)
                    plsc.store_scatter(nrm_v, [p4 + 2], nz)
                else:
                    sdf = (a * x * x + b * y * y + c * z * z
                           + d * x + e * y + f * z + g)
                    sdf_v[pl.ds(base16, L)] = sdf

        for s in range(s_max):
            base = (wid * s_max + s) * CHUNK
            pltpu.sync_copy(sidx_hbm.at[pl.ds(base, CHUNK)], idx_v)
            pltpu.sync_copy(spts_hbm.at[pl.ds(base * 4, 4 * CHUNK)], pts_v)
            compute_chunk(False)
            pltpu.sync_copy(sdf_v, sdf_out_hbm.at[pl.ds(base, CHUNK)])
            pltpu.sync_copy(ridx_hbm.at[pl.ds(base, CHUNK)], idx_v)
            pltpu.sync_copy(rpts_hbm.at[pl.ds(base * 4, 4 * CHUNK)], pts_v)
            compute_chunk(True)
            pltpu.sync_copy(nrm_v, nrm_out_hbm.at[pl.ds(base * 4, 4 * CHUNK)])

    return k


@jax.jit
def kernel(renderPointList, renderIndexList, sdfPointList, sdfIndexList,
           xLayer, yLayer, zLayer, offset):
    P = renderPointList.shape[0]
    work = NW * CHUNK
    P_pad = -(-P // work) * work
    k = _quadric_grid_sc(P_pad)
    rp4 = _tc_pad4(renderPointList, P_pad)
    sp4 = _tc_pad4(sdfPointList, P_pad)
    ridx = _tc_pad1(renderIndexList, P_pad)
    sidx = _tc_pad1(sdfIndexList, P_pad)
    off128 = jnp.repeat(offset, 2 * L)
    sdf, nrm4 = k(rp4, ridx, sp4, sidx, xLayer, yLayer, zLayer, off128)
    return _tc_slice1(sdf, P), _tc_unpack3(nrm4, P_pad, P)


# speedup vs baseline: 19.8930x; 10.8065x over previous
import functools
import jax
import jax.numpy as jnp
from jax import lax
from jax.experimental import pallas as pl
from jax.experimental.pallas import tpu as pltpu
from jax.experimental.pallas import tpu_sc as plsc

RESO = 128
L = 16
NC = 2
NS = 16
NW = NC * NS
CHUNK = 4096
GROUPS = CHUNK // L

B1 = 131072


def _tc_pad1(x, P_pad):
    P = x.shape[0]
    nin = -(-P // B1)

    def body(i_ref, o_ref):
        o_ref[...] = i_ref[...]

    return pl.pallas_call(
        body,
        grid=(P_pad // B1,),
        in_specs=[pl.BlockSpec((B1,), lambda i: (jnp.minimum(i, nin - 1),))],
        out_specs=pl.BlockSpec((B1,), lambda i: (i,)),
        out_shape=jax.ShapeDtypeStruct((P_pad,), x.dtype),
    )(x)


def _tc_slice1(x, P):
    def body(i_ref, o_ref):
        o_ref[...] = i_ref[...]

    return pl.pallas_call(
        body,
        grid=(-(-P // B1),),
        in_specs=[pl.BlockSpec((B1,), lambda i: (i,))],
        out_specs=pl.BlockSpec((B1,), lambda i: (i,)),
        out_shape=jax.ShapeDtypeStruct((P,), x.dtype),
    )(x)


def _quadric_grid_sc(P_pad):
    s_max = P_pad // (NW * CHUNK)
    mesh = plsc.VectorSubcoreMesh(core_axis_name="c", subcore_axis_name="s",
                                  num_cores=NC, num_subcores=NS)

    out1 = jax.ShapeDtypeStruct((P_pad,), jnp.float32)

    @functools.partial(
        pl.kernel,
        out_type=(out1, out1, out1, out1),
        mesh=mesh,
        compiler_params=pltpu.CompilerParams(needs_layout_passes=False),
        scratch_types=dict(
            xl=pltpu.VMEM((RESO,), jnp.float32),
            yl=pltpu.VMEM((RESO,), jnp.float32),
            zl=pltpu.VMEM((RESO,), jnp.float32),
            off=pltpu.VMEM((RESO,), jnp.float32),
            idx_v=pltpu.VMEM((CHUNK,), jnp.int32),
            px=pltpu.VMEM((CHUNK,), jnp.float32),
            py=pltpu.VMEM((CHUNK,), jnp.float32),
            pz=pltpu.VMEM((CHUNK,), jnp.float32),
            ox=pltpu.VMEM((CHUNK,), jnp.float32),
            oy=pltpu.VMEM((CHUNK,), jnp.float32),
            oz=pltpu.VMEM((CHUNK,), jnp.float32),
        ),
    )
    def k(rx_hbm, ry_hbm, rz_hbm, ridx_hbm, sx_hbm, sy_hbm, sz_hbm, sidx_hbm,
          xl_hbm, yl_hbm, zl_hbm, off_hbm,
          sdf_out, nx_out, ny_out, nz_out,
          xl, yl, zl, off, idx_v, px, py, pz, ox, oy, oz):
        wid = lax.axis_index("s") * NC + lax.axis_index("c")

        pltpu.sync_copy(xl_hbm, xl)
        pltpu.sync_copy(yl_hbm, yl)
        pltpu.sync_copy(zl_hbm, zl)
        pltpu.sync_copy(off_hbm, off)

        d = off[pl.ds(0 * 2 * L, L)]
        e = off[pl.ds(1 * 2 * L, L)]
        f = off[pl.ds(2 * 2 * L, L)]
        g = off[pl.ds(3 * 2 * L, L)]

        def tables(grp):
            base16 = grp * L
            idx = idx_v[pl.ds(base16, L)]
            ii = lax.shift_right_logical(idx, 14) & (RESO - 1)
            jj = lax.shift_right_logical(idx, 7) & (RESO - 1)
            kk = idx & (RESO - 1)
            a = plsc.load_gather(xl, [ii])
            b = plsc.load_gather(yl, [jj])
            c = plsc.load_gather(zl, [kk])
            return base16, a, b, c

        for s in range(s_max):
            base = (wid * s_max + s) * CHUNK
            pltpu.sync_copy(sidx_hbm.at[pl.ds(base, CHUNK)], idx_v)
            pltpu.sync_copy(sx_hbm.at[pl.ds(base, CHUNK)], px)
            pltpu.sync_copy(sy_hbm.at[pl.ds(base, CHUNK)], py)
            pltpu.sync_copy(sz_hbm.at[pl.ds(base, CHUNK)], pz)

            @pl.loop(0, GROUPS)
            def _(grp):
                base16, a, b, c = tables(grp)
                sl = pl.ds(base16, L)
                x = px[sl]
                y = py[sl]
                z = pz[sl]
                ox[sl] = (a * x * x + b * y * y + c * z * z
                          + d * x + e * y + f * z + g)

            pltpu.sync_copy(ox, sdf_out.at[pl.ds(base, CHUNK)])

            pltpu.sync_copy(ridx_hbm.at[pl.ds(base, CHUNK)], idx_v)
            pltpu.sync_copy(rx_hbm.at[pl.ds(base, CHUNK)], px)
            pltpu.sync_copy(ry_hbm.at[pl.ds(base, CHUNK)], py)
            pltpu.sync_copy(rz_hbm.at[pl.ds(base, CHUNK)], pz)

            @pl.loop(0, GROUPS)
            def _(grp):
                base16, a, b, c = tables(grp)
                sl = pl.ds(base16, L)
                ox[sl] = (a + a) * px[sl] + d
                oy[sl] = (b + b) * py[sl] + e
                oz[sl] = (c + c) * pz[sl] + f

            pltpu.sync_copy(ox, nx_out.at[pl.ds(base, CHUNK)])
            pltpu.sync_copy(oy, ny_out.at[pl.ds(base, CHUNK)])
            pltpu.sync_copy(oz, nz_out.at[pl.ds(base, CHUNK)])

    return k


@jax.jit
def kernel(renderPointList, renderIndexList, sdfPointList, sdfIndexList,
           xLayer, yLayer, zLayer, offset):
    P = renderPointList.shape[0]
    work = NW * CHUNK
    P_pad = -(-P // work) * work
    k = _quadric_grid_sc(P_pad)
    rx = _tc_pad1(renderPointList[:, 0], P_pad)
    ry = _tc_pad1(renderPointList[:, 1], P_pad)
    rz = _tc_pad1(renderPointList[:, 2], P_pad)
    sx = _tc_pad1(sdfPointList[:, 0], P_pad)
    sy = _tc_pad1(sdfPointList[:, 1], P_pad)
    sz = _tc_pad1(sdfPointList[:, 2], P_pad)
    ridx = _tc_pad1(renderIndexList, P_pad)
    sidx = _tc_pad1(sdfIndexList, P_pad)
    off128 = jnp.repeat(offset, 2 * L)
    sdf, nx, ny, nz = k(rx, ry, rz, ridx, sx, sy, sz, sidx,
                        xLayer, yLayer, zLayer, off128)
    nrm = jnp.stack([_tc_slice1(nx, P), _tc_slice1(ny, P),
                     _tc_slice1(nz, P)], axis=1)
    return _tc_slice1(sdf, P), nrm


# speedup vs baseline: 27.0197x; 1.3583x over previous
import functools
import jax
import jax.numpy as jnp
from jax import lax
from jax.experimental import pallas as pl
from jax.experimental.pallas import tpu as pltpu
from jax.experimental.pallas import tpu_sc as plsc

RESO = 128
L = 16
NC = 2
NS = 16
NW = NC * NS
CHUNK = 4096
GROUPS = CHUNK // L

B1 = 131072


def _tc_pad_batch(arrays, P_pad):
    P = arrays[0].shape[0]
    nin = -(-P // B1)
    n = len(arrays)

    def body(*refs):
        ins, outs = refs[:n], refs[n:]
        for i_ref, o_ref in zip(ins, outs):
            o_ref[...] = i_ref[...]

    return pl.pallas_call(
        body,
        grid=(P_pad // B1,),
        in_specs=[pl.BlockSpec((B1,), lambda i: (jnp.minimum(i, nin - 1),))
                  for _ in arrays],
        out_specs=[pl.BlockSpec((B1,), lambda i: (i,)) for _ in arrays],
        out_shape=[jax.ShapeDtypeStruct((P_pad,), a.dtype) for a in arrays],
    )(*arrays)


def _tc_slice_batch(arrays, P):
    n = len(arrays)

    def body(*refs):
        ins, outs = refs[:n], refs[n:]
        for i_ref, o_ref in zip(ins, outs):
            o_ref[...] = i_ref[...]

    return pl.pallas_call(
        body,
        grid=(-(-P // B1),),
        in_specs=[pl.BlockSpec((B1,), lambda i: (i,)) for _ in arrays],
        out_specs=[pl.BlockSpec((B1,), lambda i: (i,)) for _ in arrays],
        out_shape=[jax.ShapeDtypeStruct((P,), a.dtype) for a in arrays],
    )(*arrays)


def _quadric_grid_sc(P_pad):
    s_max = P_pad // (NW * CHUNK)
    n_units = 2 * s_max
    mesh = plsc.VectorSubcoreMesh(core_axis_name="c", subcore_axis_name="s",
                                  num_cores=NC, num_subcores=NS)

    out1 = jax.ShapeDtypeStruct((P_pad,), jnp.float32)
    buf_f = pltpu.VMEM((CHUNK,), jnp.float32)
    buf_i = pltpu.VMEM((CHUNK,), jnp.int32)

    @functools.partial(
        pl.kernel,
        out_type=(out1, out1, out1, out1),
        mesh=mesh,
        compiler_params=pltpu.CompilerParams(needs_layout_passes=False),
        scratch_types=dict(
            xl=pltpu.VMEM((RESO,), jnp.float32),
            yl=pltpu.VMEM((RESO,), jnp.float32),
            zl=pltpu.VMEM((RESO,), jnp.float32),
            off=pltpu.VMEM((RESO,), jnp.float32),
            idx_v=[buf_i, buf_i],
            px=[buf_f, buf_f],
            py=[buf_f, buf_f],
            pz=[buf_f, buf_f],
            ox=[buf_f, buf_f],
            oy=[buf_f, buf_f],
            oz=[buf_f, buf_f],
            in_sem=[pltpu.SemaphoreType.DMA, pltpu.SemaphoreType.DMA],
            out_sem=[pltpu.SemaphoreType.DMA, pltpu.SemaphoreType.DMA],
        ),
    )
    def k(rx_hbm, ry_hbm, rz_hbm, ridx_hbm, sx_hbm, sy_hbm, sz_hbm, sidx_hbm,
          xl_hbm, yl_hbm, zl_hbm, off_hbm,
          sdf_out, nx_out, ny_out, nz_out,
          xl, yl, zl, off, idx_v, px, py, pz, ox, oy, oz, in_sem, out_sem):
        wid = lax.axis_index("s") * NC + lax.axis_index("c")

        pltpu.sync_copy(xl_hbm, xl)
        pltpu.sync_copy(yl_hbm, yl)
        pltpu.sync_copy(zl_hbm, zl)
        pltpu.sync_copy(off_hbm, off)

        d = off[pl.ds(0 * 2 * L, L)]
        e = off[pl.ds(1 * 2 * L, L)]
        f = off[pl.ds(2 * 2 * L, L)]
        g = off[pl.ds(3 * 2 * L, L)]

        def unit_base(u):
            return (wid * s_max + (u // 2)) * CHUNK

        def issue_loads(u, b):
            base = unit_base(u)
            sl = pl.ds(base, CHUNK)
            if u % 2 == 0:
                srcs = (sidx_hbm, sx_hbm, sy_hbm, sz_hbm)
            else:
                srcs = (ridx_hbm, rx_hbm, ry_hbm, rz_hbm)
            dsts = (idx_v[b], px[b], py[b], pz[b])
            return [pltpu.async_copy(s.at[sl], t, in_sem[b])
                    for s, t in zip(srcs, dsts)]

        def compute(u, b):
            if u % 2 == 0:
                @pl.loop(0, GROUPS)
                def _(grp):
                    sl = pl.ds(grp * L, L)
                    idx = idx_v[b][sl]
                    ii = lax.shift_right_logical(idx, 14) & (RESO - 1)
                    jj = lax.shift_right_logical(idx, 7) & (RESO - 1)
                    kk = idx & (RESO - 1)
                    a = plsc.load_gather(xl, [ii])
                    bb = plsc.load_gather(yl, [jj])
                    c = plsc.load_gather(zl, [kk])
                    x = px[b][sl]
                    y = py[b][sl]
                    z = pz[b][sl]
                    ox[b][sl] = (a * x * x + bb * y * y + c * z * z
                                 + d * x + e * y + f * z + g)
            else:
                @pl.loop(0, GROUPS)
                def _(grp):
                    sl = pl.ds(grp * L, L)
                    idx = idx_v[b][sl]
                    ii = lax.shift_right_logical(idx, 14) & (RESO - 1)
                    jj = lax.shift_right_logical(idx, 7) & (RESO - 1)
                    kk = idx & (RESO - 1)
                    a = plsc.load_gather(xl, [ii])
                    bb = plsc.load_gather(yl, [jj])
                    c = plsc.load_gather(zl, [kk])
                    ox[b][sl] = (a + a) * px[b][sl] + d
                    oy[b][sl] = (bb + bb) * py[b][sl] + e
                    oz[b][sl] = (c + c) * pz[b][sl] + f

        def issue_stores(u, b):
            base = unit_base(u)
            sl = pl.ds(base, CHUNK)
            if u % 2 == 0:
                return [pltpu.async_copy(ox[b], sdf_out.at[sl], out_sem[b])]
            return [pltpu.async_copy(ox[b], nx_out.at[sl], out_sem[b]),
                    pltpu.async_copy(oy[b], ny_out.at[sl], out_sem[b]),
                    pltpu.async_copy(oz[b], nz_out.at[sl], out_sem[b])]

        loads = {0: issue_loads(0, 0)}
        stores = {}
        for u in range(n_units):
            b = u % 2
            if u + 1 < n_units:
                loads[u + 1] = issue_loads(u + 1, (u + 1) % 2)
            for dsc in loads.pop(u):
                dsc.wait()
            if u - 2 >= 0:
                for dsc in stores.pop(u - 2):
                    dsc.wait()
            compute(u, b)
            stores[u] = issue_stores(u, b)
        for u in (n_units - 2, n_units - 1):
            for dsc in stores.pop(u):
                dsc.wait()

    return k


@jax.jit
def kernel(renderPointList, renderIndexList, sdfPointList, sdfIndexList,
           xLayer, yLayer, zLayer, offset):
    P = renderPointList.shape[0]
    work = NW * CHUNK
    P_pad = -(-P // work) * work
    k = _quadric_grid_sc(P_pad)
    rx, ry, rz, sx, sy, sz, ridx, sidx = _tc_pad_batch(
        [renderPointList[:, 0], renderPointList[:, 1], renderPointList[:, 2],
         sdfPointList[:, 0], sdfPointList[:, 1], sdfPointList[:, 2],
         renderIndexList, sdfIndexList], P_pad)
    off128 = jnp.repeat(offset, 2 * L)
    sdf, nx, ny, nz = k(rx, ry, rz, ridx, sx, sy, sz, sidx,
                        xLayer, yLayer, zLayer, off128)
    sdf_o, nx_o, ny_o, nz_o = _tc_slice_batch([sdf, nx, ny, nz], P)
    nrm = jnp.stack([nx_o, ny_o, nz_o], axis=1)
    return sdf_o, nrm


# speedup vs baseline: 33.7206x; 1.2480x over previous
import functools
import jax
import jax.numpy as jnp
from jax import lax
from jax.experimental import pallas as pl
from jax.experimental.pallas import tpu as pltpu
from jax.experimental.pallas import tpu_sc as plsc

RESO = 128
L = 16
NC = 2
NS = 16
NW = NC * NS
CHUNK = 4096
GROUPS = CHUNK // L
B1 = 131072


def _tc_pad_batch(arrays, P_pad):
    P = arrays[0].shape[0]
    nin = -(-P // B1)
    n = len(arrays)

    def body(*refs):
        ins, outs = refs[:n], refs[n:]
        for i_ref, o_ref in zip(ins, outs):
            o_ref[...] = i_ref[...]

    return pl.pallas_call(
        body,
        grid=(P_pad // B1,),
        in_specs=[pl.BlockSpec((B1,), lambda i: (jnp.minimum(i, nin - 1),))
                  for _ in arrays],
        out_specs=[pl.BlockSpec((B1,), lambda i: (i,)) for _ in arrays],
        out_shape=[jax.ShapeDtypeStruct((P_pad,), a.dtype) for a in arrays],
    )(*arrays)


def _tc_slice_batch(arrays, P):
    n = len(arrays)

    def body(*refs):
        ins, outs = refs[:n], refs[n:]
        for i_ref, o_ref in zip(ins, outs):
            o_ref[...] = i_ref[...]

    return pl.pallas_call(
        body,
        grid=(-(-P // B1),),
        in_specs=[pl.BlockSpec((B1,), lambda i: (i,)) for _ in arrays],
        out_specs=[pl.BlockSpec((B1,), lambda i: (i,)) for _ in arrays],
        out_shape=[jax.ShapeDtypeStruct((P,), a.dtype) for a in arrays],
    )(*arrays)


def _quadric_list_sc(P_pad, want_normal):
    s_max = P_pad // (NW * CHUNK)
    mesh = plsc.VectorSubcoreMesh(core_axis_name="c", subcore_axis_name="s",
                                  num_cores=NC, num_subcores=NS)
    out1 = jax.ShapeDtypeStruct((P_pad,), jnp.float32)
    buf_f = pltpu.VMEM((CHUNK,), jnp.float32)
    buf_i = pltpu.VMEM((CHUNK,), jnp.int32)

    @functools.partial(
        pl.kernel,
        out_type=(out1, out1, out1) if want_normal else (out1,),
        mesh=mesh,
        compiler_params=pltpu.CompilerParams(needs_layout_passes=False),
        scratch_types=dict(
            xl=pltpu.VMEM((RESO,), jnp.float32),
            yl=pltpu.VMEM((RESO,), jnp.float32),
            zl=pltpu.VMEM((RESO,), jnp.float32),
            off=pltpu.VMEM((RESO,), jnp.float32),
            idx_v=[buf_i, buf_i],
            px=[buf_f, buf_f],
            py=[buf_f, buf_f],
            pz=[buf_f, buf_f],
            ox=[buf_f, buf_f],
            oy=[buf_f, buf_f],
            oz=[buf_f, buf_f],
            in_sem=[pltpu.SemaphoreType.DMA, pltpu.SemaphoreType.DMA],
            out_sem=[pltpu.SemaphoreType.DMA, pltpu.SemaphoreType.DMA],
        ),
    )
    def k(*refs, xl, yl, zl, off, idx_v, px, py, pz, ox, oy, oz,
          in_sem, out_sem):
        (x_hbm, y_hbm, z_hbm, idx_hbm, xl_hbm, yl_hbm, zl_hbm, off_hbm) = refs[:8]
        outs = refs[8:]
        wid = lax.axis_index("s") * NC + lax.axis_index("c")

        pltpu.sync_copy(xl_hbm, xl)
        pltpu.sync_copy(yl_hbm, yl)
        pltpu.sync_copy(zl_hbm, zl)
        pltpu.sync_copy(off_hbm, off)

        d = off[pl.ds(0 * 2 * L, L)]
        e = off[pl.ds(1 * 2 * L, L)]
        f = off[pl.ds(2 * 2 * L, L)]
        g = off[pl.ds(3 * 2 * L, L)]

        def issue_loads(u, b):
            sl = pl.ds((wid * s_max + u) * CHUNK, CHUNK)
            return [pltpu.async_copy(s.at[sl], t, in_sem[b])
                    for s, t in zip((idx_hbm, x_hbm, y_hbm, z_hbm),
                                    (idx_v[b], px[b], py[b], pz[b]))]

        def compute(b):
            @pl.loop(0, GROUPS)
            def _(grp):
                sl = pl.ds(grp * L, L)
                idx = idx_v[b][sl]
                ii = lax.shift_right_logical(idx, 14) & (RESO - 1)
                jj = lax.shift_right_logical(idx, 7) & (RESO - 1)
                kk = idx & (RESO - 1)
                a = plsc.load_gather(xl, [ii])
                bb = plsc.load_gather(yl, [jj])
                c = plsc.load_gather(zl, [kk])
                if want_normal:
                    ox[b][sl] = (a + a) * px[b][sl] + d
                    oy[b][sl] = (bb + bb) * py[b][sl] + e
                    oz[b][sl] = (c + c) * pz[b][sl] + f
                else:
                    x = px[b][sl]
                    y = py[b][sl]
                    z = pz[b][sl]
                    ox[b][sl] = (a * x * x + bb * y * y + c * z * z
                                 + d * x + e * y + f * z + g)

        def issue_stores(u, b):
            sl = pl.ds((wid * s_max + u) * CHUNK, CHUNK)
            srcs = (ox[b], oy[b], oz[b]) if want_normal else (ox[b],)
            return [pltpu.async_copy(s, o.at[sl], out_sem[b])
                    for s, o in zip(srcs, outs)]

        loads = {0: issue_loads(0, 0)}
        stores = {}
        for u in range(s_max):
            b = u % 2
            if u + 1 < s_max:
                loads[u + 1] = issue_loads(u + 1, (u + 1) % 2)
            for dsc in loads.pop(u):
                dsc.wait()
            if u - 2 >= 0:
                for dsc in stores.pop(u - 2):
                    dsc.wait()
            compute(b)
            stores[u] = issue_stores(u, b)
        for u in (s_max - 2, s_max - 1):
            for dsc in stores.pop(u):
                dsc.wait()

    return k


@jax.jit
def kernel(renderPointList, renderIndexList, sdfPointList, sdfIndexList,
           xLayer, yLayer, zLayer, offset):
    P = renderPointList.shape[0]
    work = NW * CHUNK
    P_pad = -(-P // work) * work
    k_sdf = _quadric_list_sc(P_pad, want_normal=False)
    k_nrm = _quadric_list_sc(P_pad, want_normal=True)
    off128 = jnp.repeat(offset, 2 * L)
    sx, sy, sz, sidx = _tc_pad_batch(
        [sdfPointList[:, 0], sdfPointList[:, 1], sdfPointList[:, 2],
         sdfIndexList], P_pad)
    (sdf,) = k_sdf(sx, sy, sz, sidx, xLayer, yLayer, zLayer, off128)
    rx, ry, rz, ridx = _tc_pad_batch(
        [renderPointList[:, 0], renderPointList[:, 1], renderPointList[:, 2],
         renderIndexList], P_pad)
    nx, ny, nz = k_nrm(rx, ry, rz, ridx, xLayer, yLayer, zLayer, off128)
    (sdf_o,) = _tc_slice_batch([sdf], P)
    nx_o, ny_o, nz_o = _tc_slice_batch([nx, ny, nz], P)
    nrm = jnp.stack([nx_o, ny_o, nz_o], axis=1)
    return sdf_o, nrm
